# baseline (speedup 1.0000x reference)
import functools

import jax
import jax.numpy as jnp
from jax import lax
from jax.experimental import pallas as pl
from jax.experimental.pallas import tpu as pltpu
from jax.experimental.pallas import tpu_sc as plsc

BATCH = 16384
DIM = 32
NC = 2
NS = 16
NW = NC * NS
BW = BATCH // NW
CH = 128
NCH = BW // CH
GROUPS = BW // 16
VOCAB4 = 250000

_MESH = plsc.VectorSubcoreMesh(
    core_axis_name="c", subcore_axis_name="s", num_cores=NC, num_subcores=NS
)


def _tables_body(x0_hbm, x1_hbm, u4_hbm, p4_hbm, dot_hbm,
                 idx0, idx1, idx0q, idx1q, uch, pch, dv, sem):
    wid = lax.axis_index("s") * NC + lax.axis_index("c")

    pltpu.sync_copy(x0_hbm.at[wid], idx0)
    pltpu.sync_copy(x1_hbm.at[wid], idx1)

    for c in range(NCH):
        for k in range(CH // 16):
            s = pl.ds(k * 16, 16)
            idx0q.at[c][s] = lax.shift_right_logical(idx0[pl.ds(c * CH + k * 16, 16)], 2)
            idx1q.at[c][s] = lax.shift_right_logical(idx1[pl.ds(c * CH + k * 16, 16)], 2)

    lane = lax.broadcasted_iota(jnp.int32, (16,), 0)

    for c in range(NCH):
        cp_u = pltpu.async_copy(u4_hbm.at[idx0q.at[c]], uch, sem)
        cp_p = pltpu.async_copy(p4_hbm.at[idx1q.at[c]], pch, sem)
        cp_u.wait()
        cp_p.wait()

        for g in range(CH // 16):
            rows = g * 16 + lane
            bsl = pl.ds(c * CH + g * 16, 16)
            colu = (idx0[bsl] & 3) * DIM
            colp = (idx1[bsl] & 3) * DIM
            acc = plsc.load_gather(uch, [rows, colu]) * plsc.load_gather(pch, [rows, colp])
            for d in range(1, DIM):
                uu = plsc.load_gather(uch, [rows, colu + d])
                pp = plsc.load_gather(pch, [rows, colp + d])
                acc = acc + uu * pp
            dv[bsl] = acc

    pltpu.sync_copy(dv, dot_hbm.at[pl.ds(wid * BW, BW)])


_tables_call = functools.partial(
    pl.kernel,
    mesh=_MESH,
    out_type=jax.ShapeDtypeStruct((BATCH,), jnp.float32),
    compiler_params=pltpu.CompilerParams(
        needs_layout_passes=False, use_tc_tiling_on_sc=True
    ),
    scratch_types=[
        pltpu.VMEM((BW,), jnp.int32),
        pltpu.VMEM((BW,), jnp.int32),
        pltpu.VMEM((NCH, CH), jnp.int32),
        pltpu.VMEM((NCH, CH), jnp.int32),
        pltpu.VMEM((CH, 128), jnp.float32),
        pltpu.VMEM((CH, 128), jnp.float32),
        pltpu.VMEM((BW,), jnp.float32),
        pltpu.SemaphoreType.DMA,
    ],
)(_tables_body)


def _bias_body(x0_hbm, x1_hbm, ub_hbm, pb_hbm, dot_hbm, out_hbm,
               idx0, idx1, ubv, pbv, dv, ov, sem):
    wid = lax.axis_index("s") * NC + lax.axis_index("c")

    pltpu.sync_copy(x0_hbm.at[wid], idx0)
    pltpu.sync_copy(x1_hbm.at[wid], idx1)
    pltpu.sync_copy(dot_hbm.at[pl.ds(wid * BW, BW)], dv)

    copies = []
    for c in range(NCH):
        dst = pl.ds(c * CH, CH)
        copies.append(pltpu.async_copy(ub_hbm.at[0].at[idx0.at[c]], ubv.at[dst], sem))
        copies.append(pltpu.async_copy(pb_hbm.at[0].at[idx1.at[c]], pbv.at[dst], sem))
    for cp in copies:
        cp.wait()

    def group(g, carry):
        base = g * 16
        acc = dv[pl.ds(base, 16)] + ubv[pl.ds(base, 16)] + pbv[pl.ds(base, 16)]
        ov[pl.ds(base, 16)] = 1.05 / (1.0 + jnp.exp(-acc))
        return carry

    lax.fori_loop(0, GROUPS, group, 0)
    pltpu.sync_copy(ov, out_hbm.at[pl.ds(wid * BW, BW)])


_bias_call = functools.partial(
    pl.kernel,
    mesh=_MESH,
    out_type=jax.ShapeDtypeStruct((BATCH,), jnp.float32),
    compiler_params=pltpu.CompilerParams(
        needs_layout_passes=False, use_tc_tiling_on_sc=False
    ),
    scratch_types=[
        pltpu.VMEM((NCH, CH), jnp.int32),
        pltpu.VMEM((NCH, CH), jnp.int32),
        pltpu.VMEM((BW,), jnp.float32),
        pltpu.VMEM((BW,), jnp.float32),
        pltpu.VMEM((BW,), jnp.float32),
        pltpu.VMEM((BW,), jnp.float32),
        pltpu.SemaphoreType.DMA,
    ],
)(_bias_body)


def kernel(x, u, u_bias, p, p_bias):
    x0 = x[:, 0].astype(jnp.int32)
    x1 = x[:, 1].astype(jnp.int32)
    dot = _tables_call(
        x0.reshape(NW, BW), x1.reshape(NW, BW),
        u.reshape(VOCAB4, 128), p.reshape(VOCAB4, 128),
    )
    return _bias_call(
        x0.reshape(NW, NCH, CH), x1.reshape(NW, NCH, CH),
        u_bias.T, p_bias.T, dot,
    )
